# baseline (device time: 115352 ns/iter reference)
import jax
import jax.numpy as jnp
from jax import lax
from jax.experimental import pallas as pl
from jax.experimental.pallas import tpu as pltpu

N_DEV = 8


def kernel(x, w_mat, scale_x, scale_w):
    x8 = x.astype(jnp.float8_e4m3fn)
    w8 = w_mat.astype(jnp.float8_e4m3fn)
    s = (scale_x.astype(jnp.float32) * scale_w.astype(jnp.float32)).reshape(1, 1)

    m_per, k = x.shape
    n_per = w_mat.shape[1]
    m_half = m_per // 2
    m_out = N_DEV * m_per

    def body(x_ref, w_ref, s_ref, out_ref,
             buf_a, buf_b, send_a, recv_a, send_b, recv_b):
        my = lax.axis_index("i")
        left = (my + N_DEV - 1) % N_DEV
        right = (my + 1) % N_DEV

        barrier = pltpu.get_barrier_semaphore()
        for nbr in (left, right):
            pl.semaphore_signal(
                barrier, inc=1,
                device_id=(nbr,), device_id_type=pl.DeviceIdType.MESH,
            )
        pl.semaphore_wait(barrier, 2)

        scale = s_ref[0, 0]

        def mm_silu(a):
            acc = lax.dot_general(
                a, w_ref[...],
                (((1,), (0,)), ((), ())),
                preferred_element_type=jnp.float32,
            )
            y = acc * scale
            return y * jax.nn.sigmoid(y)

        buf_a[0] = x_ref[0:m_half, :]
        buf_b[0] = x_ref[m_half:m_per, :]
        out_ref[pl.ds(my * m_per, m_per), :] = mm_silu(x_ref[...])

        for h in range(N_DEV - 1):
            ra = pltpu.make_async_remote_copy(
                src_ref=buf_a.at[h], dst_ref=buf_a.at[h + 1],
                send_sem=send_a.at[h], recv_sem=recv_a.at[h],
                device_id=(right,), device_id_type=pl.DeviceIdType.MESH,
            )
            rb = pltpu.make_async_remote_copy(
                src_ref=buf_b.at[h], dst_ref=buf_b.at[h + 1],
                send_sem=send_b.at[h], recv_sem=recv_b.at[h],
                device_id=(left,), device_id_type=pl.DeviceIdType.MESH,
            )
            ra.start()
            rb.start()
            ra.wait()
            rb.wait()
            oa = (my + N_DEV - 1 - h) % N_DEV
            ob = (my + 1 + h) % N_DEV
            out_ref[pl.ds(oa * m_per, m_half), :] = mm_silu(buf_a[h + 1])
            out_ref[pl.ds(ob * m_per + m_half, m_half), :] = mm_silu(buf_b[h + 1])

    return pl.pallas_call(
        body,
        out_shape=jax.ShapeDtypeStruct((m_out, n_per), jnp.float32),
        in_specs=[
            pl.BlockSpec(memory_space=pltpu.VMEM),
            pl.BlockSpec(memory_space=pltpu.VMEM),
            pl.BlockSpec(memory_space=pltpu.SMEM),
        ],
        out_specs=pl.BlockSpec(memory_space=pltpu.VMEM),
        scratch_shapes=[
            pltpu.VMEM((N_DEV, m_half, k), jnp.float8_e4m3fn),
            pltpu.VMEM((N_DEV, m_half, k), jnp.float8_e4m3fn),
            pltpu.SemaphoreType.DMA((N_DEV - 1,)),
            pltpu.SemaphoreType.DMA((N_DEV - 1,)),
            pltpu.SemaphoreType.DMA((N_DEV - 1,)),
            pltpu.SemaphoreType.DMA((N_DEV - 1,)),
        ],
        compiler_params=pltpu.CompilerParams(collective_id=0),
    )(x8, w8, s)


# device time: 108032 ns/iter; 1.0678x vs baseline; 1.0678x over previous
import jax
import jax.numpy as jnp
from jax import lax
from jax.experimental import pallas as pl
from jax.experimental.pallas import tpu as pltpu

N_DEV = 8


def kernel(x, w_mat, scale_x, scale_w):
    x8 = x.astype(jnp.float8_e4m3fn)
    w8 = w_mat.astype(jnp.float8_e4m3fn)
    s = (scale_x.astype(jnp.float32) * scale_w.astype(jnp.float32)).reshape(1, 1)

    m_per, k = x.shape
    n_per = w_mat.shape[1]
    m_half = m_per // 2
    m_out = N_DEV * m_per

    def body(x_ref, w_ref, s_ref, out_ref,
             buf_a, buf_b, send_a, recv_a, send_b, recv_b):
        my = lax.axis_index("i")
        left = (my + N_DEV - 1) % N_DEV
        right = (my + 1) % N_DEV

        barrier = pltpu.get_barrier_semaphore()
        for nbr in (left, right):
            pl.semaphore_signal(
                barrier, inc=1,
                device_id=(nbr,), device_id_type=pl.DeviceIdType.MESH,
            )
        pl.semaphore_wait(barrier, 2)

        scale = s_ref[0, 0]

        def mm_silu(a):
            acc = lax.dot_general(
                a, w_ref[...],
                (((1,), (0,)), ((), ())),
                preferred_element_type=jnp.float32,
            )
            y = acc * scale
            return y * jax.nn.sigmoid(y)

        def make_hop(h):
            ra = pltpu.make_async_remote_copy(
                src_ref=buf_a.at[h], dst_ref=buf_a.at[h + 1],
                send_sem=send_a.at[h], recv_sem=recv_a.at[h],
                device_id=(right,), device_id_type=pl.DeviceIdType.MESH,
            )
            rb = pltpu.make_async_remote_copy(
                src_ref=buf_b.at[h], dst_ref=buf_b.at[h + 1],
                send_sem=send_b.at[h], recv_sem=recv_b.at[h],
                device_id=(left,), device_id_type=pl.DeviceIdType.MESH,
            )
            return ra, rb

        buf_a[0] = x_ref[0:m_half, :]
        buf_b[0] = x_ref[m_half:m_per, :]
        ra, rb = make_hop(0)
        ra.start()
        rb.start()
        out_ref[pl.ds(my * m_per, m_per), :] = mm_silu(x_ref[...])

        for h in range(N_DEV - 1):
            ra.wait()
            rb.wait()
            if h < N_DEV - 2:
                ra, rb = make_hop(h + 1)
                ra.start()
                rb.start()
            oa = (my + N_DEV - 1 - h) % N_DEV
            ob = (my + 1 + h) % N_DEV
            out_ref[pl.ds(oa * m_per, m_half), :] = mm_silu(buf_a[h + 1])
            out_ref[pl.ds(ob * m_per + m_half, m_half), :] = mm_silu(buf_b[h + 1])

    return pl.pallas_call(
        body,
        out_shape=jax.ShapeDtypeStruct((m_out, n_per), jnp.float32),
        in_specs=[
            pl.BlockSpec(memory_space=pltpu.VMEM),
            pl.BlockSpec(memory_space=pltpu.VMEM),
            pl.BlockSpec(memory_space=pltpu.SMEM),
        ],
        out_specs=pl.BlockSpec(memory_space=pltpu.VMEM),
        scratch_shapes=[
            pltpu.VMEM((N_DEV, m_half, k), jnp.float8_e4m3fn),
            pltpu.VMEM((N_DEV, m_half, k), jnp.float8_e4m3fn),
            pltpu.SemaphoreType.DMA((N_DEV - 1,)),
            pltpu.SemaphoreType.DMA((N_DEV - 1,)),
            pltpu.SemaphoreType.DMA((N_DEV - 1,)),
            pltpu.SemaphoreType.DMA((N_DEV - 1,)),
        ],
        compiler_params=pltpu.CompilerParams(collective_id=0),
    )(x8, w8, s)


# device time: 95414 ns/iter; 1.2090x vs baseline; 1.1322x over previous
import jax
import jax.numpy as jnp
from jax import lax
from jax.experimental import pallas as pl
from jax.experimental.pallas import tpu as pltpu

N_DEV = 8
N_SEG = 2


def kernel(x, w_mat, scale_x, scale_w):
    x8 = x.astype(jnp.float8_e4m3fn)
    w8 = w_mat.astype(jnp.float8_e4m3fn)
    s = (scale_x.astype(jnp.float32) * scale_w.astype(jnp.float32)).reshape(1, 1)

    m_per, k = x.shape
    n_per = w_mat.shape[1]
    m_half = m_per // 2
    m_seg = m_half // N_SEG
    m_out = N_DEV * m_per

    def body(x_ref, w_ref, s_ref, out_ref,
             buf_a, buf_b, send_a, recv_a, send_b, recv_b):
        my = lax.axis_index("i")
        left = (my + N_DEV - 1) % N_DEV
        right = (my + 1) % N_DEV

        barrier = pltpu.get_barrier_semaphore()
        for nbr in (left, right):
            pl.semaphore_signal(
                barrier, inc=1,
                device_id=(nbr,), device_id_type=pl.DeviceIdType.MESH,
            )
        pl.semaphore_wait(barrier, 2)

        scale = s_ref[0, 0]

        def mm_silu(a):
            acc = lax.dot_general(
                a, w_ref[...],
                (((1,), (0,)), ((), ())),
                preferred_element_type=jnp.float32,
            )
            y = acc * scale
            return y * jax.nn.sigmoid(y)

        def make_hop(h, seg):
            ra = pltpu.make_async_remote_copy(
                src_ref=buf_a.at[h, seg], dst_ref=buf_a.at[h + 1, seg],
                send_sem=send_a.at[h, seg], recv_sem=recv_a.at[h, seg],
                device_id=(right,), device_id_type=pl.DeviceIdType.MESH,
            )
            rb = pltpu.make_async_remote_copy(
                src_ref=buf_b.at[h, seg], dst_ref=buf_b.at[h + 1, seg],
                send_sem=send_b.at[h, seg], recv_sem=recv_b.at[h, seg],
                device_id=(left,), device_id_type=pl.DeviceIdType.MESH,
            )
            return ra, rb

        buf_a[0] = x_ref[0:m_half, :].reshape(N_SEG, m_seg, k)
        buf_b[0] = x_ref[m_half:m_per, :].reshape(N_SEG, m_seg, k)
        hop = [make_hop(0, seg) for seg in range(N_SEG)]
        for ra, rb in hop:
            ra.start()
            rb.start()
        out_ref[pl.ds(my * m_per, m_per), :] = mm_silu(x_ref[...])

        for h in range(N_DEV - 1):
            nxt = []
            for seg in range(N_SEG):
                ra, rb = hop[seg]
                ra.wait()
                rb.wait()
                if h < N_DEV - 2:
                    ra, rb = make_hop(h + 1, seg)
                    ra.start()
                    rb.start()
                    nxt.append((ra, rb))
            hop = nxt
            oa = (my + N_DEV - 1 - h) % N_DEV
            ob = (my + 1 + h) % N_DEV
            out_ref[pl.ds(oa * m_per, m_half), :] = mm_silu(
                buf_a[h + 1].reshape(m_half, k))
            out_ref[pl.ds(ob * m_per + m_half, m_half), :] = mm_silu(
                buf_b[h + 1].reshape(m_half, k))

    return pl.pallas_call(
        body,
        out_shape=jax.ShapeDtypeStruct((m_out, n_per), jnp.float32),
        in_specs=[
            pl.BlockSpec(memory_space=pltpu.VMEM),
            pl.BlockSpec(memory_space=pltpu.VMEM),
            pl.BlockSpec(memory_space=pltpu.SMEM),
        ],
        out_specs=pl.BlockSpec(memory_space=pltpu.VMEM),
        scratch_shapes=[
            pltpu.VMEM((N_DEV, N_SEG, m_seg, k), jnp.float8_e4m3fn),
            pltpu.VMEM((N_DEV, N_SEG, m_seg, k), jnp.float8_e4m3fn),
            pltpu.SemaphoreType.DMA((N_DEV - 1, N_SEG)),
            pltpu.SemaphoreType.DMA((N_DEV - 1, N_SEG)),
            pltpu.SemaphoreType.DMA((N_DEV - 1, N_SEG)),
            pltpu.SemaphoreType.DMA((N_DEV - 1, N_SEG)),
        ],
        compiler_params=pltpu.CompilerParams(collective_id=0),
    )(x8, w8, s)
